# Initial kernel scaffold; baseline (speedup 1.0000x reference)
import jax
import jax.numpy as jnp
from jax.experimental import pallas as pl


def kernel(h_s, p_s, bs, h_v, p_v, bv, q_s, q_v, role_s, role_v, normal_s, normal_v, dipole_s, dipole_v, cross_edge_index, params):
    raise NotImplementedError("write your pallas kernel here")


# speedup vs baseline: 8.4876x; 8.4876x over previous
import functools
import math

import jax
import jax.numpy as jnp
from jax import lax
from jax.experimental import pallas as pl
from jax.experimental.pallas import tpu as pltpu, tpu_sc as plsc

_CUTOFF = 4.5
_TD = 160
_SR = 144
_CH = 128


def _sig(x):
    return 1.0 / (1.0 + jnp.exp(-x))


def _silu(x):
    return x * _sig(x)


def _softplus(x):
    return jnp.maximum(x, 0.0) + jnp.log(1.0 + jnp.exp(-jnp.abs(x)))


def _node_table_body(h_ref, sm_ref, w1_ref, b1_ref, w2_ref, b2_ref, out_ref):
    h = h_ref[...]
    t = _silu(jnp.dot(h, w1_ref[...], preferred_element_type=jnp.float32)
              + b1_ref[...])
    t2 = jnp.dot(t, w2_ref[...], preferred_element_type=jnp.float32) + b2_ref[...]
    lj = _softplus(t2[:, 0:2]) + 1e-3
    n = h.shape[0]
    out_ref[...] = jnp.concatenate(
        [h, sm_ref[...], lj, jnp.zeros((n, _TD - 145), jnp.float32)], axis=1)


def _node_tables(h_all, small_all, p):
    n2, hid = h_all.shape
    blk = 2000 if n2 % 2000 == 0 else n2
    w2p = jnp.pad(p['phys_W2'], ((0, 0), (0, 6)))
    b2p = jnp.pad(p['phys_b2'], (0, 6)).reshape(1, 8)
    return pl.pallas_call(
        _node_table_body,
        grid=(n2 // blk,),
        in_specs=[
            pl.BlockSpec((blk, hid), lambda i: (i, 0)),
            pl.BlockSpec((blk, 15), lambda i: (i, 0)),
            pl.BlockSpec((hid, hid), lambda i: (0, 0)),
            pl.BlockSpec((1, hid), lambda i: (0, 0)),
            pl.BlockSpec((hid, 8), lambda i: (0, 0)),
            pl.BlockSpec((1, 8), lambda i: (0, 0)),
        ],
        out_specs=pl.BlockSpec((blk, _TD), lambda i: (i, 0)),
        out_shape=jax.ShapeDtypeStruct((n2, _TD), jnp.float32),
    )(h_all, small_all, p['phys_W1'], p['phys_b1'].reshape(1, hid), w2p, b2p)


def _gather(ts, tv, idx_s, idx_v):
    ne = idx_s.shape[0]
    nchunk = ne // _CH
    info = plsc.get_sparse_core_info()
    nc, nsub = info.num_cores, info.num_subcores
    nw = nc * nsub
    jmax = (nchunk + nw - 1) // nw
    mesh = plsc.VectorSubcoreMesh(core_axis_name="c", subcore_axis_name="s")

    @functools.partial(
        pl.kernel, mesh=mesh,
        compiler_params=pltpu.CompilerParams(use_tc_tiling_on_sc=False),
        out_type=(jax.ShapeDtypeStruct((ne, _TD), jnp.float32),
                  jax.ShapeDtypeStruct((ne, _TD), jnp.float32)),
        scratch_types=[
            pltpu.VMEM((_CH,), jnp.int32),
            pltpu.VMEM((_CH,), jnp.int32),
            pltpu.VMEM((_CH, _TD), jnp.float32),
            pltpu.VMEM((_CH, _TD), jnp.float32),
            pltpu.SemaphoreType.DMA,
            pltpu.SemaphoreType.DMA,
        ])
    def gk(ts_h, tv_h, is_h, iv_h, os_h, ov_h, ib_s, ib_v, rb_s, rb_v, sa, sb):
        wid = lax.axis_index("s") * nc + lax.axis_index("c")

        def body(j, carry):
            g = j * nw + wid

            @pl.when(g < nchunk)
            def _():
                base = g * _CH
                pltpu.sync_copy(is_h.at[pl.ds(base, _CH)], ib_s)
                pltpu.sync_copy(iv_h.at[pl.ds(base, _CH)], ib_v)
                a = pltpu.async_copy(ts_h.at[ib_s], rb_s, sa)
                b = pltpu.async_copy(tv_h.at[ib_v], rb_v, sb)
                a.wait()
                b.wait()
                pltpu.sync_copy(rb_s, os_h.at[pl.ds(base, _CH)])
                pltpu.sync_copy(rb_v, ov_h.at[pl.ds(base, _CH)])
            return carry

        lax.fori_loop(0, jmax, body, 0)

    return gk(ts, tv, idx_s, idx_v)


def _edge_body(rs_ref, rv_ref, w1b_ref, b1b_ref, gw2_ref, gb2_ref, ew2_ref,
               eb2_ref, tw1_ref, tb1_ref, tw2_ref, tb2_ref, eng_ref, enb_ref,
               w2b_ref, b2b_ref, cw2_ref, cb2_ref, os_ref, ov_ref, *, hid, nr,
               nexp, ed):
    rs = rs_ref[...]
    rv = rv_ref[...]
    e = rs.shape[0]
    pp_ = e // 128
    hs = rs[:, 0:hid]
    hv = rv[:, 0:hid]

    ss = jnp.transpose(rs[:, hid:hid + 32]).reshape(32, pp_, 128)
    vv = jnp.transpose(rv[:, hid:hid + 32]).reshape(32, pp_, 128)
    relx = ss[0] - vv[0]
    rely = ss[1] - vv[1]
    relz = ss[2] - vv[2]
    dist = jnp.sqrt(relx * relx + rely * rely + relz * relz + 1e-8)
    safe = jnp.maximum(dist, 0.5)
    inv_safe = 1.0 / safe
    ux = relx * inv_safe
    uy = rely * inv_safe
    uz = relz * inv_safe
    eps_ij = jnp.sqrt(ss[15] * vv[15] + 1e-8)
    ratio = jnp.minimum(0.5 * (ss[16] + vv[16]) * inv_safe, 1.5)
    r2 = ratio * ratio
    r6 = r2 * r2 * r2
    e_lj = 4.0 * eps_ij * (r6 * r6 - r6)
    cp = ss[3] * vv[3]
    e_coul = cp * inv_safe
    cd = jnp.abs(ss[3] - vv[3])
    da = ss[4] * vv[5] + ss[5] * vv[4]
    ar = ss[6] * vv[6]
    ndot = ss[9] * vv[9] + ss[10] * vv[10] + ss[11] * vv[11]
    na = ar * jnp.abs(ndot)
    ddot = ss[12] * vv[12] + ss[13] * vv[13] + ss[14] * vv[14]
    dop = 1.0 - jnp.abs(ddot)
    hp = ss[7] * vv[7]
    ppair = ss[8] * vv[8]
    hpp = ss[7] * vv[8] + ss[8] * vv[7]

    clip = jnp.maximum(dist, 1e-6)
    rclip = 1.0 / clip
    theta = (math.pi / _CUTOFF) * clip
    s1 = jnp.sin(theta)
    twoc = 2.0 * jnp.cos(theta)
    rad = [s1, twoc * s1]
    for _ in range(nr - 2):
        rad.append(twoc * rad[-1] - rad[-2])
    rows = ([r * rclip for r in rad]
            + [dist * (1.0 / _CUTOFF), inv_safe, cp, cd, e_lj, e_coul,
               da, na, hp, da, ar, na, ddot, dop, hp, ppair, hpp])
    tt = jnp.transpose(jnp.stack(rows).reshape(nr + 17, e))
    itype = tt[:, nr + 9:nr + 17]
    th = _silu(jnp.dot(
        jnp.concatenate([itype, jnp.zeros((e, 8), jnp.float32)], axis=1),
        tw1_ref[...], preferred_element_type=jnp.float32) + tb1_ref[...])
    th = jnp.dot(th, tw2_ref[...], preferred_element_type=jnp.float32) + tb2_ref[...]

    esp = w1b_ref.shape[0]
    es = jnp.concatenate(
        [hs, hv, tt[:, 0:nr + 9], th, jnp.zeros((e, esp - ed), jnp.float32)],
        axis=1)
    sums = jnp.dot(jnp.concatenate([es, es * es], axis=0),
                   jnp.ones((esp, 8), jnp.float32),
                   preferred_element_type=jnp.float32)
    mu = sums[0:e, 0:1] * (1.0 / ed)
    msq = sums[e:2 * e, 0:1] * (1.0 / ed)
    var = msq - mu * mu
    esn = (es - mu) * lax.rsqrt(var + 1e-5) * eng_ref[...] + enb_ref[...]

    h1 = jnp.dot(esn.astype(jnp.bfloat16), w1b_ref[...],
                 preferred_element_type=jnp.float32) + b1b_ref[...]
    a = _silu(h1)
    gl = jnp.dot(a[:, 0:hid], gw2_ref[...], preferred_element_type=jnp.float32) \
        + gb2_ref[...]
    gl = gl[:, 0:nexp]
    gmax = jnp.max(gl, axis=1, keepdims=True)
    ge = jnp.exp(gl - gmax)
    gw = ge / jnp.sum(ge, axis=1, keepdims=True)

    m = jnp.zeros((e, hid), jnp.float32)
    for k in range(nexp):
        ak = a[:, hid * (k + 1):hid * (k + 2)].astype(jnp.bfloat16)
        hk = _silu(jnp.dot(ak, ew2_ref[hid * k:hid * (k + 1), :],
                           preferred_element_type=jnp.float32)
                   + eb2_ref[k:k + 1, :])
        m = m + gw[:, k:k + 1] * hk

    mo = jnp.dot(m.astype(jnp.bfloat16), w2b_ref[...],
                 preferred_element_type=jnp.float32) + b2b_ref[...]
    hh = hid // 2
    wsv = jnp.dot(_silu(mo[:, 2 * hid:2 * hid + 2 * hh]), cw2_ref[...],
                  preferred_element_type=jnp.float32) + cb2_ref[...]
    wp = jnp.transpose(wsv).reshape(2, pp_, 128)
    one_p = jnp.ones((pp_, 128), jnp.float32)
    tails = jnp.transpose(jnp.stack(
        [wp[0] * ux, wp[0] * uy, wp[0] * uz, one_p,
         -wp[1] * ux, -wp[1] * uy, -wp[1] * uz, one_p]).reshape(8, e))
    zpad = jnp.zeros((e, _SR - 132), jnp.float32)
    os_ref[...] = jnp.concatenate([mo[:, 0:hid], tails[:, 0:4], zpad], axis=1)
    ov_ref[...] = jnp.concatenate([mo[:, hid:2 * hid], tails[:, 4:8], zpad],
                                  axis=1)


def _edge_compute(es_rows, ev_rows, p):
    ne = es_rows.shape[0]
    hid = p['phys_W1'].shape[0]
    hh = hid // 2
    ed = p['en_g'].shape[0]
    nr = ed - 2 * hid - 9 - hh
    nexp = p['exp_W1'].shape[0]
    esp = ((ed + 127) // 128) * 128
    eblk = 1280 if ne % 1280 == 0 else (640 if ne % 640 == 0 else ne)

    w1b = jnp.pad(
        jnp.concatenate([p['gate_W1']] + [p['exp_W1'][k] for k in range(nexp)],
                        axis=1), ((0, esp - ed), (0, 0))).astype(jnp.bfloat16)
    b1b = jnp.concatenate([p['gate_b1']] + [p['exp_b1'][k] for k in range(nexp)]
                          ).reshape(1, -1)
    gw2 = jnp.pad(p['gate_W2'], ((0, 0), (0, 8 - nexp)))
    gb2 = jnp.pad(p['gate_b2'], (0, 8 - nexp)).reshape(1, 8)
    ew2 = jnp.concatenate([p['exp_W2'][k] for k in range(nexp)],
                          axis=0).astype(jnp.bfloat16)
    eb2 = p['exp_b2']
    tw1 = jnp.pad(p['type_W1'], ((0, 8), (0, 0)))
    tb1 = p['type_b1'].reshape(1, hh)
    tw2 = p['type_W2']
    tb2 = p['type_b2'].reshape(1, hh)
    eng = jnp.pad(p['en_g'], (0, esp - ed)).reshape(1, esp)
    enb = jnp.pad(p['en_b'], (0, esp - ed)).reshape(1, esp)
    w2b = jnp.concatenate([p['msgv_W'], p['msgs_W'], p['cs_W1'], p['cv_W1']],
                          axis=1).astype(jnp.bfloat16)
    b2b = jnp.concatenate([p['msgv_b'], p['msgs_b'], p['cs_b1'], p['cv_b1']]
                          ).reshape(1, -1)
    cw2 = jnp.concatenate([
        jnp.concatenate([p['cs_W2'], jnp.zeros((hh, 1), jnp.float32)], axis=1),
        jnp.concatenate([jnp.zeros((hh, 1), jnp.float32), p['cv_W2']], axis=1),
    ], axis=0)
    cb2 = jnp.concatenate([p['cs_b2'], p['cv_b2']]).reshape(1, 2)

    nb = 2 * hid + 2 * hh
    body = functools.partial(_edge_body, hid=hid, nr=nr, nexp=nexp, ed=ed)
    wspec = lambda w: pl.BlockSpec(w.shape, lambda i: tuple(0 for _ in w.shape))
    weights = [w1b, b1b, gw2, gb2, ew2, eb2, tw1, tb1, tw2, tb2, eng, enb,
               w2b, b2b, cw2, cb2]
    return pl.pallas_call(
        body,
        grid=(ne // eblk,),
        in_specs=[pl.BlockSpec((eblk, _TD), lambda i: (i, 0)),
                  pl.BlockSpec((eblk, _TD), lambda i: (i, 0))]
        + [wspec(w) for w in weights],
        out_specs=[pl.BlockSpec((eblk, _SR), lambda i: (i, 0)),
                   pl.BlockSpec((eblk, _SR), lambda i: (i, 0))],
        out_shape=[jax.ShapeDtypeStruct((ne, _SR), jnp.float32),
                   jax.ShapeDtypeStruct((ne, _SR), jnp.float32)],
    )(es_rows, ev_rows, *weights)


def _scatter(out_s, out_v, idx_s, idx_v, ns, nv):
    ne = idx_s.shape[0]
    nchunk = ne // _CH
    info = plsc.get_sparse_core_info()
    nsub = info.num_subcores
    jmax = (nchunk + nsub - 1) // nsub
    rpt = ns // nsub
    mesh = plsc.VectorSubcoreMesh(core_axis_name="c", subcore_axis_name="s")
    zz = jnp.zeros((ns, _SR), jnp.float32)

    @functools.partial(
        pl.kernel, mesh=mesh,
        compiler_params=pltpu.CompilerParams(use_tc_tiling_on_sc=False),
        out_type=(jax.ShapeDtypeStruct((ns, _SR), jnp.float32),
                  jax.ShapeDtypeStruct((nv, _SR), jnp.float32)),
        scratch_types=[
            pltpu.VMEM((_CH,), jnp.int32),
            pltpu.VMEM((_CH, _SR), jnp.float32),
            pltpu.VMEM_SHARED((ns, _SR), jnp.float32),
        ])
    def sk(rs_h, rv_h, is_h, iv_h, zz_h, os_h, ov_h, ib, rb, acc):
        c = lax.axis_index("c")
        sid = lax.axis_index("s")
        r0 = sid * rpt
        pltpu.sync_copy(zz_h.at[pl.ds(r0, rpt)], acc.at[pl.ds(r0, rpt)])
        plsc.subcore_barrier()

        def side(rows_h, idx_h):
            def body(j, carry):
                g = j * nsub + sid

                @pl.when(g < nchunk)
                def _():
                    base = g * _CH
                    pltpu.sync_copy(idx_h.at[pl.ds(base, _CH)], ib)
                    pltpu.sync_copy(rows_h.at[pl.ds(base, _CH)], rb)
                    pltpu.sync_copy(rb, acc.at[ib], add=True)
                return carry

            lax.fori_loop(0, jmax, body, 0)

        @pl.when(c == 0)
        def _():
            side(rs_h, is_h)

        @pl.when(c == 1)
        def _():
            side(rv_h, iv_h)

        plsc.subcore_barrier()

        @pl.when(c == 0)
        def _():
            pltpu.sync_copy(acc.at[pl.ds(r0, rpt)], os_h.at[pl.ds(r0, rpt)])

        @pl.when(c == 1)
        def _():
            pltpu.sync_copy(acc.at[pl.ds(r0, rpt)], ov_h.at[pl.ds(r0, rpt)])

    return sk(out_s, out_v, idx_s, idx_v, zz)


def _update_body(hs_ref, ps_ref, ss_ref, hv_ref, pv_ref, sv_ref,
                 usw_ref, usb_ref, usg_ref, usbeta_ref,
                 uvw_ref, uvb_ref, uvg_ref, uvbeta_ref,
                 hso_ref, pso_ref, hvo_ref, pvo_ref):
    def one(h_ref, p_ref, s_ref, w_ref, b_ref, g_ref, beta_ref, ho_ref, po_ref):
        h = h_ref[...]
        s = s_ref[...]
        hid = h.shape[1]
        rc = 1.0 / jnp.maximum(s[:, 131:132], 1.0)
        agg = s[:, 0:hid] * rc
        x = jnp.dot(jnp.concatenate([h, agg], axis=1), w_ref[...],
                    preferred_element_type=jnp.float32) + b_ref[...]
        y = _silu(x)
        mu = jnp.mean(y, axis=1, keepdims=True)
        yc = y - mu
        var = jnp.mean(yc * yc, axis=1, keepdims=True)
        ho_ref[...] = h + yc * lax.rsqrt(var + 1e-5) * g_ref[...] + beta_ref[...]
        po_ref[...] = p_ref[...] + s[:, hid:hid + 3] * rc

    one(hs_ref, ps_ref, ss_ref, usw_ref, usb_ref, usg_ref, usbeta_ref,
        hso_ref, pso_ref)
    one(hv_ref, pv_ref, sv_ref, uvw_ref, uvb_ref, uvg_ref, uvbeta_ref,
        hvo_ref, pvo_ref)


def _node_update(h_s, p_s, sum_s, h_v, p_v, sum_v, p):
    n, hid = h_s.shape
    blk = 2000 if n % 2000 == 0 else n
    ws = [p['ups_W'], p['ups_b'].reshape(1, hid), p['ups_g'].reshape(1, hid),
          p['ups_beta'].reshape(1, hid),
          p['upv_W'], p['upv_b'].reshape(1, hid), p['upv_g'].reshape(1, hid),
          p['upv_beta'].reshape(1, hid)]
    wspec = lambda w: pl.BlockSpec(w.shape, lambda i: tuple(0 for _ in w.shape))
    return pl.pallas_call(
        _update_body,
        grid=(n // blk,),
        in_specs=[pl.BlockSpec((blk, hid), lambda i: (i, 0)),
                  pl.BlockSpec((blk, 3), lambda i: (i, 0)),
                  pl.BlockSpec((blk, _SR), lambda i: (i, 0)),
                  pl.BlockSpec((blk, hid), lambda i: (i, 0)),
                  pl.BlockSpec((blk, 3), lambda i: (i, 0)),
                  pl.BlockSpec((blk, _SR), lambda i: (i, 0))]
        + [wspec(w) for w in ws],
        out_specs=[pl.BlockSpec((blk, hid), lambda i: (i, 0)),
                   pl.BlockSpec((blk, 3), lambda i: (i, 0)),
                   pl.BlockSpec((blk, hid), lambda i: (i, 0)),
                   pl.BlockSpec((blk, 3), lambda i: (i, 0))],
        out_shape=[jax.ShapeDtypeStruct((n, hid), jnp.float32),
                   jax.ShapeDtypeStruct((n, 3), jnp.float32),
                   jax.ShapeDtypeStruct((n, hid), jnp.float32),
                   jax.ShapeDtypeStruct((n, 3), jnp.float32)],
    )(h_s, p_s, sum_s, h_v, p_v, sum_v, *ws)


def kernel(h_s, p_s, bs, h_v, p_v, bv, q_s, q_v, role_s, role_v, normal_s,
           normal_v, dipole_s, dipole_v, cross_edge_index, params):
    ns = h_s.shape[0]
    nv = h_v.shape[0]
    idx_s = cross_edge_index[0]
    idx_v = cross_edge_index[1]
    small_s = jnp.concatenate([p_s, q_s, role_s, normal_s, dipole_s], axis=1)
    small_v = jnp.concatenate([p_v, q_v, role_v, normal_v, dipole_v], axis=1)
    h_all = jnp.concatenate([h_s, h_v], axis=0)
    small_all = jnp.concatenate([small_s, small_v], axis=0)
    table = _node_tables(h_all, small_all, params)
    ts = table[:ns]
    tv = table[ns:]
    es_rows, ev_rows = _gather(ts, tv, idx_s, idx_v)
    out_s, out_v = _edge_compute(es_rows, ev_rows, params)
    sum_s, sum_v = _scatter(out_s, out_v, idx_s, idx_v, ns, nv)
    return _node_update(h_s, p_s, sum_s, h_v, p_v, sum_v, params)


# speedup vs baseline: 11.3137x; 1.3330x over previous
import functools
import math

import jax
import jax.numpy as jnp
from jax import lax
from jax.experimental import pallas as pl
from jax.experimental.pallas import tpu as pltpu, tpu_sc as plsc

_CUTOFF = 4.5
_TD = 256
_CH = 128


def _sig(x):
    return 1.0 / (1.0 + jnp.exp(-x))


def _silu(x):
    return x * _sig(x)


def _softplus(x):
    return jnp.maximum(x, 0.0) + jnp.log(1.0 + jnp.exp(-jnp.abs(x)))


def _node_table_body(h_ref, sm_ref, w1_ref, b1_ref, w2_ref, b2_ref, out_ref):
    h = h_ref[...]
    t = _silu(jnp.dot(h, w1_ref[...], preferred_element_type=jnp.float32)
              + b1_ref[...])
    t2 = jnp.dot(t, w2_ref[...], preferred_element_type=jnp.float32) + b2_ref[...]
    lj = _softplus(t2[:, 0:2]) + 1e-3
    n = h.shape[0]
    out_ref[...] = jnp.concatenate(
        [h, sm_ref[...], lj, jnp.zeros((n, _TD - 145), jnp.float32)], axis=1)


def _node_tables(h, small, p):
    n, hid = h.shape
    blk = 2000 if n % 2000 == 0 else n
    w2p = jnp.pad(p['phys_W2'], ((0, 0), (0, 6)))
    b2p = jnp.pad(p['phys_b2'], (0, 6)).reshape(1, 8)
    return pl.pallas_call(
        _node_table_body,
        grid=(n // blk,),
        in_specs=[
            pl.BlockSpec((blk, hid), lambda i: (i, 0)),
            pl.BlockSpec((blk, 15), lambda i: (i, 0)),
            pl.BlockSpec((hid, hid), lambda i: (0, 0)),
            pl.BlockSpec((1, hid), lambda i: (0, 0)),
            pl.BlockSpec((hid, 8), lambda i: (0, 0)),
            pl.BlockSpec((1, 8), lambda i: (0, 0)),
        ],
        out_specs=pl.BlockSpec((blk, _TD), lambda i: (i, 0)),
        out_shape=jax.ShapeDtypeStruct((n, _TD), jnp.float32),
    )(h, small, p['phys_W1'], p['phys_b1'].reshape(1, hid), w2p, b2p)


def _gather(ts, tv, idx_s, idx_v):
    ne = idx_s.shape[0]
    nchunk = ne // _CH
    info = plsc.get_sparse_core_info()
    nc, nsub = info.num_cores, info.num_subcores
    nw = nc * nsub
    jmax = (nchunk + nw - 1) // nw
    mesh = plsc.VectorSubcoreMesh(core_axis_name="c", subcore_axis_name="s")

    @functools.partial(
        pl.kernel, mesh=mesh,
        out_type=(jax.ShapeDtypeStruct((ne, _TD), jnp.float32),
                  jax.ShapeDtypeStruct((ne, _TD), jnp.float32)),
        scratch_types=[
            pltpu.VMEM((_CH,), jnp.int32),
            pltpu.VMEM((_CH,), jnp.int32),
            pltpu.VMEM((_CH, _TD), jnp.float32),
            pltpu.VMEM((_CH, _TD), jnp.float32),
            pltpu.SemaphoreType.DMA,
            pltpu.SemaphoreType.DMA,
        ])
    def gk(ts_h, tv_h, is_h, iv_h, os_h, ov_h, ib_s, ib_v, rb_s, rb_v, sa, sb):
        wid = lax.axis_index("s") * nc + lax.axis_index("c")

        def body(j, carry):
            g = j * nw + wid

            @pl.when(g < nchunk)
            def _():
                base = g * _CH
                pltpu.sync_copy(is_h.at[pl.ds(base, _CH)], ib_s)
                pltpu.sync_copy(iv_h.at[pl.ds(base, _CH)], ib_v)
                a = pltpu.async_copy(ts_h.at[ib_s], rb_s, sa)
                b = pltpu.async_copy(tv_h.at[ib_v], rb_v, sb)
                a.wait()
                b.wait()
                pltpu.sync_copy(rb_s, os_h.at[pl.ds(base, _CH)])
                pltpu.sync_copy(rb_v, ov_h.at[pl.ds(base, _CH)])
            return carry

        lax.fori_loop(0, jmax, body, 0)

    return gk(ts, tv, idx_s, idx_v)


def _edge_body(rs_ref, rv_ref, w1b_ref, b1b_ref, gw2_ref, gb2_ref, ew2_ref,
               eb2_ref, tw1_ref, tb1_ref, tw2_ref, tb2_ref, eng_ref, enb_ref,
               w2b_ref, b2b_ref, cw2_ref, cb2_ref, os_ref, ov_ref, ot_ref,
               *, hid, nr, nexp, ed):
    rs = rs_ref[...]
    rv = rv_ref[...]
    e = rs.shape[0]
    pp_ = e // 128
    hs = rs[:, 0:hid]
    hv = rv[:, 0:hid]

    ss = jnp.transpose(rs[:, hid:hid + 32]).reshape(32, pp_, 128)
    vv = jnp.transpose(rv[:, hid:hid + 32]).reshape(32, pp_, 128)
    relx = ss[0] - vv[0]
    rely = ss[1] - vv[1]
    relz = ss[2] - vv[2]
    dist = jnp.sqrt(relx * relx + rely * rely + relz * relz + 1e-8)
    safe = jnp.maximum(dist, 0.5)
    inv_safe = 1.0 / safe
    ux = relx * inv_safe
    uy = rely * inv_safe
    uz = relz * inv_safe
    eps_ij = jnp.sqrt(ss[15] * vv[15] + 1e-8)
    ratio = jnp.minimum(0.5 * (ss[16] + vv[16]) * inv_safe, 1.5)
    r2 = ratio * ratio
    r6 = r2 * r2 * r2
    e_lj = 4.0 * eps_ij * (r6 * r6 - r6)
    cp = ss[3] * vv[3]
    e_coul = cp * inv_safe
    cd = jnp.abs(ss[3] - vv[3])
    da = ss[4] * vv[5] + ss[5] * vv[4]
    ar = ss[6] * vv[6]
    ndot = ss[9] * vv[9] + ss[10] * vv[10] + ss[11] * vv[11]
    na = ar * jnp.abs(ndot)
    ddot = ss[12] * vv[12] + ss[13] * vv[13] + ss[14] * vv[14]
    dop = 1.0 - jnp.abs(ddot)
    hp = ss[7] * vv[7]
    ppair = ss[8] * vv[8]
    hpp = ss[7] * vv[8] + ss[8] * vv[7]

    clip = jnp.maximum(dist, 1e-6)
    rclip = 1.0 / clip
    theta = (math.pi / _CUTOFF) * clip
    s1 = jnp.sin(theta)
    twoc = 2.0 * jnp.cos(theta)
    rad = [s1, twoc * s1]
    for _ in range(nr - 2):
        rad.append(twoc * rad[-1] - rad[-2])
    rows = ([r * rclip for r in rad]
            + [dist * (1.0 / _CUTOFF), inv_safe, cp, cd, e_lj, e_coul,
               da, na, hp, da, ar, na, ddot, dop, hp, ppair, hpp])
    tt = jnp.transpose(jnp.stack(rows).reshape(nr + 17, e))
    itype = tt[:, nr + 9:nr + 17]
    th = _silu(jnp.dot(
        jnp.concatenate([itype, jnp.zeros((e, 8), jnp.float32)], axis=1),
        tw1_ref[...], preferred_element_type=jnp.float32) + tb1_ref[...])
    th = jnp.dot(th, tw2_ref[...], preferred_element_type=jnp.float32) + tb2_ref[...]

    esp = w1b_ref.shape[0]
    es = jnp.concatenate(
        [hs, hv, tt[:, 0:nr + 9], th, jnp.zeros((e, esp - ed), jnp.float32)],
        axis=1)
    sums = jnp.dot(jnp.concatenate([es, es * es], axis=0),
                   jnp.ones((esp, 8), jnp.float32),
                   preferred_element_type=jnp.float32)
    mu = sums[0:e, 0:1] * (1.0 / ed)
    msq = sums[e:2 * e, 0:1] * (1.0 / ed)
    var = msq - mu * mu
    esn = (es - mu) * lax.rsqrt(var + 1e-5) * eng_ref[...] + enb_ref[...]

    h1 = jnp.dot(esn.astype(jnp.bfloat16), w1b_ref[...],
                 preferred_element_type=jnp.float32) + b1b_ref[...]
    a = _silu(h1)
    gl = jnp.dot(a[:, 0:hid], gw2_ref[...], preferred_element_type=jnp.float32) \
        + gb2_ref[...]
    gl = gl[:, 0:nexp]
    gmax = jnp.max(gl, axis=1, keepdims=True)
    ge = jnp.exp(gl - gmax)
    gw = ge / jnp.sum(ge, axis=1, keepdims=True)

    m = jnp.zeros((e, hid), jnp.float32)
    for k in range(nexp):
        ak = a[:, hid * (k + 1):hid * (k + 2)].astype(jnp.bfloat16)
        hk = _silu(jnp.dot(ak, ew2_ref[hid * k:hid * (k + 1), :],
                           preferred_element_type=jnp.float32)
                   + eb2_ref[k:k + 1, :])
        m = m + gw[:, k:k + 1] * hk

    mo = jnp.dot(m.astype(jnp.bfloat16), w2b_ref[...],
                 preferred_element_type=jnp.float32) + b2b_ref[...]
    hh = hid // 2
    wsv = jnp.dot(_silu(mo[:, 2 * hid:2 * hid + 2 * hh]), cw2_ref[...],
                  preferred_element_type=jnp.float32) + cb2_ref[...]
    wp = jnp.transpose(wsv).reshape(2, pp_, 128)
    one_p = jnp.ones((pp_, 128), jnp.float32)
    os_ref[...] = mo[:, 0:hid]
    ov_ref[...] = mo[:, hid:2 * hid]
    ot_ref[...] = jnp.stack(
        [wp[0] * ux, wp[0] * uy, wp[0] * uz, one_p,
         -wp[1] * ux, -wp[1] * uy, -wp[1] * uz, one_p]).reshape(8, e)


def _edge_compute(es_rows, ev_rows, p):
    ne = es_rows.shape[0]
    hid = p['phys_W1'].shape[0]
    hh = hid // 2
    ed = p['en_g'].shape[0]
    nr = ed - 2 * hid - 9 - hh
    nexp = p['exp_W1'].shape[0]
    esp = ((ed + 127) // 128) * 128
    eblk = 1280 if ne % 1280 == 0 else (640 if ne % 640 == 0 else ne)

    w1b = jnp.pad(
        jnp.concatenate([p['gate_W1']] + [p['exp_W1'][k] for k in range(nexp)],
                        axis=1), ((0, esp - ed), (0, 0))).astype(jnp.bfloat16)
    b1b = jnp.concatenate([p['gate_b1']] + [p['exp_b1'][k] for k in range(nexp)]
                          ).reshape(1, -1)
    gw2 = jnp.pad(p['gate_W2'], ((0, 0), (0, 8 - nexp)))
    gb2 = jnp.pad(p['gate_b2'], (0, 8 - nexp)).reshape(1, 8)
    ew2 = jnp.concatenate([p['exp_W2'][k] for k in range(nexp)],
                          axis=0).astype(jnp.bfloat16)
    eb2 = p['exp_b2']
    tw1 = jnp.pad(p['type_W1'], ((0, 8), (0, 0)))
    tb1 = p['type_b1'].reshape(1, hh)
    tw2 = p['type_W2']
    tb2 = p['type_b2'].reshape(1, hh)
    eng = jnp.pad(p['en_g'], (0, esp - ed)).reshape(1, esp)
    enb = jnp.pad(p['en_b'], (0, esp - ed)).reshape(1, esp)
    w2b = jnp.concatenate([p['msgv_W'], p['msgs_W'], p['cs_W1'], p['cv_W1']],
                          axis=1).astype(jnp.bfloat16)
    b2b = jnp.concatenate([p['msgv_b'], p['msgs_b'], p['cs_b1'], p['cv_b1']]
                          ).reshape(1, -1)
    cw2 = jnp.concatenate([
        jnp.concatenate([p['cs_W2'], jnp.zeros((hh, 1), jnp.float32)], axis=1),
        jnp.concatenate([jnp.zeros((hh, 1), jnp.float32), p['cv_W2']], axis=1),
    ], axis=0)
    cb2 = jnp.concatenate([p['cs_b2'], p['cv_b2']]).reshape(1, 2)

    body = functools.partial(_edge_body, hid=hid, nr=nr, nexp=nexp, ed=ed)
    wspec = lambda w: pl.BlockSpec(w.shape, lambda i: tuple(0 for _ in w.shape))
    weights = [w1b, b1b, gw2, gb2, ew2, eb2, tw1, tb1, tw2, tb2, eng, enb,
               w2b, b2b, cw2, cb2]
    return pl.pallas_call(
        body,
        grid=(ne // eblk,),
        in_specs=[pl.BlockSpec((eblk, _TD), lambda i: (i, 0)),
                  pl.BlockSpec((eblk, _TD), lambda i: (i, 0))]
        + [wspec(w) for w in weights],
        out_specs=[pl.BlockSpec((eblk, hid), lambda i: (i, 0)),
                   pl.BlockSpec((eblk, hid), lambda i: (i, 0)),
                   pl.BlockSpec((8, eblk), lambda i: (0, i))],
        out_shape=[jax.ShapeDtypeStruct((ne, hid), jnp.float32),
                   jax.ShapeDtypeStruct((ne, hid), jnp.float32),
                   jax.ShapeDtypeStruct((8, ne), jnp.float32)],
    )(es_rows, ev_rows, *weights)


def _scatter(msg_s, msg_v, tails, idx_s, idx_v, ns, nv):
    ne = idx_s.shape[0]
    hid = msg_s.shape[1]
    nchunk = ne // _CH
    info = plsc.get_sparse_core_info()
    nsub = info.num_subcores
    jmax = (nchunk + nsub - 1) // nsub
    rpt = (ns // nsub) // 8 * 8
    rlast = ns - rpt * (nsub - 1)
    mesh = plsc.VectorSubcoreMesh(core_axis_name="c", subcore_axis_name="s")
    zz = jnp.zeros((ns, hid), jnp.float32)
    zt = jnp.zeros((2 * ns,), jnp.float32)

    @functools.partial(
        pl.kernel, mesh=mesh,
        compiler_params=pltpu.CompilerParams(needs_layout_passes=False),
        out_type=(jax.ShapeDtypeStruct((ns, hid), jnp.float32),
                  jax.ShapeDtypeStruct((nv, hid), jnp.float32),
                  jax.ShapeDtypeStruct((nsub * 4 * ns,), jnp.float32),
                  jax.ShapeDtypeStruct((nsub * 4 * nv,), jnp.float32)),
        scratch_types=[
            pltpu.VMEM((_CH,), jnp.int32),
            pltpu.VMEM((_CH, hid), jnp.float32),
            pltpu.VMEM((8, _CH), jnp.float32),
            pltpu.VMEM((2 * ns,), jnp.float32),
            pltpu.VMEM_SHARED((ns, hid), jnp.float32),
        ])
    def sk(ms_h, mv_h, tl_h, is_h, iv_h, zz_h, zt_h, os_h, ov_h, ts_h, tv_h,
           ib, rb, tb, tacc, acc):
        c = lax.axis_index("c")
        sid = lax.axis_index("s")
        r0 = sid * rpt

        @pl.when(sid < nsub - 1)
        def _():
            pltpu.sync_copy(zz_h.at[pl.ds(r0, rpt)], acc.at[pl.ds(r0, rpt)])

        @pl.when(sid == nsub - 1)
        def _():
            pltpu.sync_copy(zz_h.at[pl.ds(r0, rlast)], acc.at[pl.ds(r0, rlast)])

        plsc.subcore_barrier()

        def side(rows_h, idx_h, trow0, t_out):
            for pp in range(2):
                pltpu.sync_copy(zt_h, tacc)

                def body(j, carry):
                    g = j * nsub + sid

                    @pl.when(g < nchunk)
                    def _():
                        base = g * _CH
                        pltpu.sync_copy(idx_h.at[pl.ds(base, _CH)], ib)
                        pltpu.sync_copy(tl_h.at[:, pl.ds(base, _CH)], tb)
                        if pp == 0:
                            pltpu.sync_copy(rows_h.at[pl.ds(base, _CH)], rb)
                            pltpu.sync_copy(rb, acc.at[ib], add=True)
                        for grp in range(8):
                            iv = ib[pl.ds(grp * 16, 16)]
                            for j2 in range(2):
                                val = tb[trow0 + 2 * pp + j2,
                                         pl.ds(grp * 16, 16)]
                                plsc.addupdate_scatter(
                                    tacc, [iv + j2 * ns], val)
                    return carry

                lax.fori_loop(0, jmax, body, 0)
                pltpu.sync_copy(
                    tacc,
                    t_out.at[pl.ds((pp * nsub + sid) * 2 * ns, 2 * ns)])

        @pl.when(c == 0)
        def _():
            side(ms_h, is_h, 0, ts_h)

        @pl.when(c == 1)
        def _():
            side(mv_h, iv_h, 4, tv_h)

        plsc.subcore_barrier()

        def dump(o_h):
            @pl.when(sid < nsub - 1)
            def _():
                pltpu.sync_copy(acc.at[pl.ds(r0, rpt)], o_h.at[pl.ds(r0, rpt)])

            @pl.when(sid == nsub - 1)
            def _():
                pltpu.sync_copy(acc.at[pl.ds(r0, rlast)],
                                o_h.at[pl.ds(r0, rlast)])

        @pl.when(c == 0)
        def _():
            dump(os_h)

        @pl.when(c == 1)
        def _():
            dump(ov_h)

    return sk(msg_s, msg_v, tails, idx_s, idx_v, zz, zt)


def _update_body(hs_ref, ps_ref, ms_ref, ts_ref, hv_ref, pv_ref, mv_ref,
                 tv_ref, rsel_ref, usw_ref, usb_ref, usg_ref, usbeta_ref,
                 uvw_ref, uvb_ref, uvg_ref, uvbeta_ref,
                 hso_ref, pso_ref, hvo_ref, pvo_ref):
    def one(h_ref, p_ref, m_ref, t_ref, w_ref, b_ref, g_ref, beta_ref,
            ho_ref, po_ref):
        h = h_ref[...]
        msum = m_ref[...]
        t4 = jnp.dot(t_ref[...], rsel_ref[...],
                     preferred_element_type=jnp.float32)
        wxy = t4[:, 0:2]
        wzc = t4[:, 2:4]
        rc = 1.0 / jnp.maximum(wzc[:, 1:2], 1.0)
        agg = msum * rc
        x = jnp.dot(jnp.concatenate([h, agg], axis=1), w_ref[...],
                    preferred_element_type=jnp.float32) + b_ref[...]
        y = _silu(x)
        mu = jnp.mean(y, axis=1, keepdims=True)
        yc = y - mu
        var = jnp.mean(yc * yc, axis=1, keepdims=True)
        ho_ref[...] = h + yc * lax.rsqrt(var + 1e-5) * g_ref[...] + beta_ref[...]
        po_ref[...] = p_ref[...] + jnp.concatenate(
            [wxy, wzc[:, 0:1]], axis=1) * rc

    one(hs_ref, ps_ref, ms_ref, ts_ref, usw_ref, usb_ref, usg_ref, usbeta_ref,
        hso_ref, pso_ref)
    one(hv_ref, pv_ref, mv_ref, tv_ref, uvw_ref, uvb_ref, uvg_ref, uvbeta_ref,
        hvo_ref, pvo_ref)


def _node_update(h_s, p_s, msum_s, traw_s, h_v, p_v, msum_v, traw_v, p):
    n, hid = h_s.shape
    blk = 2000 if n % 2000 == 0 else n
    ts3 = jnp.transpose(traw_s.reshape(2, 16, 2, n), (3, 0, 1, 2)).reshape(n, 64)
    tv3 = jnp.transpose(traw_v.reshape(2, 16, 2, n), (3, 0, 1, 2)).reshape(n, 64)
    rsel = jnp.zeros((2, 16, 2, 8), jnp.float32)
    for pp in range(2):
        for j in range(2):
            rsel = rsel.at[pp, :, j, 2 * pp + j].set(1.0)
    rsel = rsel.reshape(64, 8)
    ws = [p['ups_W'], p['ups_b'].reshape(1, hid), p['ups_g'].reshape(1, hid),
          p['ups_beta'].reshape(1, hid),
          p['upv_W'], p['upv_b'].reshape(1, hid), p['upv_g'].reshape(1, hid),
          p['upv_beta'].reshape(1, hid)]
    wspec = lambda w: pl.BlockSpec(w.shape, lambda i: tuple(0 for _ in w.shape))
    return pl.pallas_call(
        _update_body,
        grid=(n // blk,),
        in_specs=[pl.BlockSpec((blk, hid), lambda i: (i, 0)),
                  pl.BlockSpec((blk, 3), lambda i: (i, 0)),
                  pl.BlockSpec((blk, hid), lambda i: (i, 0)),
                  pl.BlockSpec((blk, 64), lambda i: (i, 0)),
                  pl.BlockSpec((blk, hid), lambda i: (i, 0)),
                  pl.BlockSpec((blk, 3), lambda i: (i, 0)),
                  pl.BlockSpec((blk, hid), lambda i: (i, 0)),
                  pl.BlockSpec((blk, 64), lambda i: (i, 0)),
                  pl.BlockSpec((64, 8), lambda i: (0, 0))]
        + [wspec(w) for w in ws],
        out_specs=[pl.BlockSpec((blk, hid), lambda i: (i, 0)),
                   pl.BlockSpec((blk, 3), lambda i: (i, 0)),
                   pl.BlockSpec((blk, hid), lambda i: (i, 0)),
                   pl.BlockSpec((blk, 3), lambda i: (i, 0))],
        out_shape=[jax.ShapeDtypeStruct((n, hid), jnp.float32),
                   jax.ShapeDtypeStruct((n, 3), jnp.float32),
                   jax.ShapeDtypeStruct((n, hid), jnp.float32),
                   jax.ShapeDtypeStruct((n, 3), jnp.float32)],
    )(h_s, p_s, msum_s, ts3, h_v, p_v, msum_v, tv3, rsel, *ws)


def kernel(h_s, p_s, bs, h_v, p_v, bv, q_s, q_v, role_s, role_v, normal_s,
           normal_v, dipole_s, dipole_v, cross_edge_index, params):
    ns = h_s.shape[0]
    nv = h_v.shape[0]
    idx_s = cross_edge_index[0]
    idx_v = cross_edge_index[1]
    small_s = jnp.concatenate([p_s, q_s, role_s, normal_s, dipole_s], axis=1)
    small_v = jnp.concatenate([p_v, q_v, role_v, normal_v, dipole_v], axis=1)
    ts = _node_tables(h_s, small_s, params)
    tv = _node_tables(h_v, small_v, params)
    es_rows, ev_rows = _gather(ts, tv, idx_s, idx_v)
    msg_s, msg_v, tails = _edge_compute(es_rows, ev_rows, params)
    msum_s, msum_v, traw_s, traw_v = _scatter(msg_s, msg_v, tails, idx_s,
                                              idx_v, ns, nv)
    return _node_update(h_s, p_s, msum_s, traw_s, h_v, p_v, msum_v, traw_v,
                        params)
